# Initial kernel scaffold; baseline (speedup 1.0000x reference)
import jax
import jax.numpy as jnp
from jax.experimental import pallas as pl


def kernel(output, label_id):
    raise NotImplementedError("write your pallas kernel here")


# speedup vs baseline: 8.8393x; 8.8393x over previous
import functools

import jax
import jax.numpy as jnp
from jax import lax
from jax.experimental import pallas as pl
from jax.experimental.pallas import tpu as pltpu
from jax.experimental.pallas import tpu_sc as plsc

_N = 65536
_D = 128
_C = 1000
_MARGIN = 1.0

_NC = 2
_NS = 16
_NW = _NC * _NS
_ROWS_PER_W = _N // _NW
_CHUNK = 128
_NCHUNK = _ROWS_PER_W // _CHUNK
_CPAD = _NS * 64


def _sc_body(out_hbm, lab_hbm, zacc_hbm, zcnt_hbm, ones_hbm,
             sums_hbm, cnts_hbm,
             idx_v, rows_v, ones_v, acc_sh, cnt_sh):
    c = lax.axis_index("c")
    s = lax.axis_index("s")
    wid = c * _NS + s
    base_row = wid * _ROWS_PER_W

    pltpu.sync_copy(zacc_hbm.at[pl.ds(s * 64, 64)], acc_sh.at[pl.ds(s * 64, 64)])
    pltpu.sync_copy(zcnt_hbm.at[pl.ds(s * 64, 64)], cnt_sh.at[pl.ds(s * 64, 64)])
    pltpu.sync_copy(ones_hbm, ones_v)
    plsc.subcore_barrier()

    for j in range(_NCHUNK):
        pltpu.sync_copy(lab_hbm.at[pl.ds(base_row + j * _CHUNK, _CHUNK)], idx_v)
        pltpu.sync_copy(out_hbm.at[pl.ds(base_row + j * _CHUNK, _CHUNK)], rows_v)
        pltpu.sync_copy(rows_v, acc_sh.at[idx_v], add=True)
        pltpu.sync_copy(ones_v, cnt_sh.at[idx_v], add=True)

    plsc.subcore_barrier()
    out_base = c * _CPAD + s * 64
    pltpu.sync_copy(acc_sh.at[pl.ds(s * 64, 64)], sums_hbm.at[pl.ds(out_base, 64)])
    pltpu.sync_copy(cnt_sh.at[pl.ds(s * 64, 64)], cnts_hbm.at[pl.ds(out_base, 64)])


_sc_segsum = functools.partial(
    pl.kernel,
    out_type=(
        jax.ShapeDtypeStruct((_NC * _CPAD, _D), jnp.float32),
        jax.ShapeDtypeStruct((_NC * _CPAD, _D), jnp.float32),
    ),
    mesh=plsc.VectorSubcoreMesh(core_axis_name="c", subcore_axis_name="s"),
    scratch_types=[
        pltpu.VMEM((_CHUNK,), jnp.int32),
        pltpu.VMEM((_CHUNK, _D), jnp.float32),
        pltpu.VMEM((_CHUNK, _D), jnp.float32),
        pltpu.VMEM_SHARED((_CPAD, _D), jnp.float32),
        pltpu.VMEM_SHARED((_CPAD, _D), jnp.float32),
    ],
)(_sc_body)


def _tc_body(sums_ref, cnts_ref, out_ref):
    sums = sums_ref[...]
    seg = sums[:_CPAD] + sums[_CPAD:]
    cnts = cnts_ref[...]
    cnt = cnts[:_CPAD, 0:1] + cnts[_CPAD:, 0:1]
    total = jnp.sum(seg, axis=0, keepdims=True)
    n = jnp.float32(_N)
    csafe = jnp.where(cnt > 0, cnt, 1.0)
    diff = seg / csafe - (total - seg) / (n - csafe) + 1e-6
    dist = jnp.sqrt(jnp.sum(diff * diff, axis=1, keepdims=True))
    term = jnp.square(jnp.maximum(_MARGIN - dist, 0.0))
    loss = jnp.sum(jnp.where(cnt > 0, cnt * term, 0.0)) / n
    out_ref[...] = jnp.reshape(loss, (1, 1))


def kernel(output, label_id):
    zacc = jnp.zeros((_CPAD, _D), jnp.float32)
    zcnt = jnp.zeros((_CPAD, _D), jnp.float32)
    ones = jnp.ones((_CHUNK, _D), jnp.float32)
    sums, cnts = _sc_segsum(output, label_id, zacc, zcnt, ones)
    loss = pl.pallas_call(
        _tc_body,
        out_shape=jax.ShapeDtypeStruct((1, 1), jnp.float32),
    )(sums, cnts)
    return loss.reshape(())


# speedup vs baseline: 15.5563x; 1.7599x over previous
import functools

import jax
import jax.numpy as jnp
from jax import lax
from jax.experimental import pallas as pl
from jax.experimental.pallas import tpu as pltpu
from jax.experimental.pallas import tpu_sc as plsc

_N = 65536
_D = 128
_C = 1000
_MARGIN = 1.0

_NC = 2
_NS = 16
_NW = _NC * _NS
_ROWS_PER_W = _N // _NW
_CHUNK = 128
_NCHUNK = _ROWS_PER_W // _CHUNK
_CPAD = _NS * 64
_L = 16


def _sc_body(out_hbm, lab_hbm, zacc_hbm, sums_hbm, cnts_hbm,
             idx_v0, idx_v1, rows_v0, rows_v1, cnt_v, acc_sh,
             load_sem, scat_sem):
    idx_b = (idx_v0, idx_v1)
    rows_b = (rows_v0, rows_v1)
    c = lax.axis_index("c")
    s = lax.axis_index("s")
    wid = c * _NS + s
    base_row = wid * _ROWS_PER_W

    zeros = jnp.zeros((_L,), jnp.float32)
    ones = jnp.ones((_L,), jnp.float32)

    for k in range(_CPAD // _L):
        cnt_v[pl.ds(k * _L, _L)] = zeros
    pltpu.sync_copy(zacc_hbm.at[pl.ds(s * 64, 64)], acc_sh.at[pl.ds(s * 64, 64)])
    plsc.subcore_barrier()

    loads = [None] * _NCHUNK
    scats = [None] * _NCHUNK

    def start_load(j):
        b = j % 2
        r0 = base_row + j * _CHUNK
        di = pltpu.async_copy(lab_hbm.at[pl.ds(r0, _CHUNK)], idx_b[b], load_sem)
        dr = pltpu.async_copy(out_hbm.at[pl.ds(r0, _CHUNK)], rows_b[b], load_sem)
        loads[j] = (di, dr)

    start_load(0)
    for j in range(_NCHUNK):
        b = j % 2
        if j >= 1:
            scats[j - 1].wait()
        if j + 1 < _NCHUNK:
            start_load(j + 1)
        di, dr = loads[j]
        di.wait()
        dr.wait()
        scats[j] = pltpu.async_copy(rows_b[b], acc_sh.at[idx_b[b]],
                                    scat_sem, add=True)
        for k in range(_CHUNK // _L):
            lbl = idx_b[b][pl.ds(k * _L, _L)]
            plsc.addupdate_scatter(cnt_v, [lbl], ones)
    scats[_NCHUNK - 1].wait()

    plsc.subcore_barrier()
    out_base = c * _CPAD + s * 64
    pltpu.sync_copy(acc_sh.at[pl.ds(s * 64, 64)], sums_hbm.at[pl.ds(out_base, 64)])
    pltpu.sync_copy(cnt_v, cnts_hbm.at[wid])


_sc_segsum = functools.partial(
    pl.kernel,
    out_type=(
        jax.ShapeDtypeStruct((_NC * _CPAD, _D), jnp.float32),
        jax.ShapeDtypeStruct((_NW, _CPAD), jnp.float32),
    ),
    mesh=plsc.VectorSubcoreMesh(core_axis_name="c", subcore_axis_name="s"),
    compiler_params=pltpu.CompilerParams(needs_layout_passes=False),
    scratch_types=[
        pltpu.VMEM((_CHUNK,), jnp.int32),
        pltpu.VMEM((_CHUNK,), jnp.int32),
        pltpu.VMEM((_CHUNK, _D), jnp.float32),
        pltpu.VMEM((_CHUNK, _D), jnp.float32),
        pltpu.VMEM((_CPAD,), jnp.float32),
        pltpu.VMEM_SHARED((_CPAD, _D), jnp.float32),
        pltpu.SemaphoreType.DMA,
        pltpu.SemaphoreType.DMA,
    ],
)(_sc_body)


def _tc_body(sums_ref, cnts_ref, out_ref):
    sums = sums_ref[...]
    seg = sums[:_CPAD] + sums[_CPAD:]
    cnt = jnp.sum(cnts_ref[...], axis=0)[:, None]
    total = jnp.sum(seg, axis=0, keepdims=True)
    n = jnp.float32(_N)
    csafe = jnp.where(cnt > 0, cnt, 1.0)
    diff = seg / csafe - (total - seg) / (n - csafe) + 1e-6
    dist = jnp.sqrt(jnp.sum(diff * diff, axis=1, keepdims=True))
    term = jnp.square(jnp.maximum(_MARGIN - dist, 0.0))
    loss = jnp.sum(jnp.where(cnt > 0, cnt * term, 0.0)) / n
    out_ref[...] = jnp.reshape(loss, (1, 1))


def kernel(output, label_id):
    zacc = jnp.zeros((_CPAD, _D), jnp.float32)
    sums, cnts = _sc_segsum(output, label_id, zacc)
    loss = pl.pallas_call(
        _tc_body,
        out_shape=jax.ShapeDtypeStruct((1, 1), jnp.float32),
    )(sums, cnts)
    return loss.reshape(())


# speedup vs baseline: 16.3090x; 1.0484x over previous
import functools

import jax
import jax.numpy as jnp
from jax import lax
from jax.experimental import pallas as pl
from jax.experimental.pallas import tpu as pltpu
from jax.experimental.pallas import tpu_sc as plsc

_N = 65536
_D = 128
_C = 1000
_MARGIN = 1.0

_NC = 2
_NS = 16
_NW = _NC * _NS
_ROWS_PER_W = _N // _NW
_CHUNK = 128
_NCHUNK = _ROWS_PER_W // _CHUNK
_CPAD = _NS * 64
_L = 16
_NBUF = 4


def _sc_body(out_hbm, lab_hbm, sums_hbm, cnts_hbm,
             idx_v0, idx_v1, idx_v2, idx_v3,
             rows_v0, rows_v1, rows_v2, rows_v3,
             cnt_v, acc_sh, load_sem, scat_sem):
    idx_b = (idx_v0, idx_v1, idx_v2, idx_v3)
    rows_b = (rows_v0, rows_v1, rows_v2, rows_v3)
    c = lax.axis_index("c")
    s = lax.axis_index("s")
    wid = c * _NS + s
    base_row = wid * _ROWS_PER_W

    zeros = jnp.zeros((_L,), jnp.float32)
    ones = jnp.ones((_L,), jnp.float32)

    for k in range(_CPAD // _L):
        cnt_v[pl.ds(k * _L, _L)] = zeros
    for r in range(64):
        for k in range(_D // _L):
            rows_v0[r, pl.ds(k * _L, _L)] = zeros
    pltpu.sync_copy(rows_v0.at[pl.ds(0, 64)], acc_sh.at[pl.ds(s * 64, 64)])
    plsc.subcore_barrier()

    loads = [None] * _NCHUNK
    scats = [None] * _NCHUNK

    def start_load(j):
        b = j % _NBUF
        r0 = base_row + j * _CHUNK
        di = pltpu.async_copy(lab_hbm.at[pl.ds(r0, _CHUNK)], idx_b[b], load_sem)
        dr = pltpu.async_copy(out_hbm.at[pl.ds(r0, _CHUNK)], rows_b[b], load_sem)
        loads[j] = (di, dr)

    for j in range(_NBUF - 1):
        start_load(j)
    for j in range(_NCHUNK):
        b = j % _NBUF
        if j + _NBUF - 1 < _NCHUNK:
            if j >= 1:
                scats[j - 1].wait()
            start_load(j + _NBUF - 1)
        di, dr = loads[j]
        di.wait()
        dr.wait()
        scats[j] = pltpu.async_copy(rows_b[b], acc_sh.at[idx_b[b]],
                                    scat_sem, add=True)
        for k in range(_CHUNK // _L):
            lbl = idx_b[b][pl.ds(k * _L, _L)]
            plsc.addupdate_scatter(cnt_v, [lbl], ones)
    for j in range(_NCHUNK - _NBUF + 1, _NCHUNK):
        scats[j].wait()

    plsc.subcore_barrier()
    out_base = c * _CPAD + s * 64
    pltpu.sync_copy(acc_sh.at[pl.ds(s * 64, 64)], sums_hbm.at[pl.ds(out_base, 64)])
    pltpu.sync_copy(cnt_v, cnts_hbm.at[wid])


_sc_segsum = functools.partial(
    pl.kernel,
    out_type=(
        jax.ShapeDtypeStruct((_NC * _CPAD, _D), jnp.float32),
        jax.ShapeDtypeStruct((_NW, _CPAD), jnp.float32),
    ),
    mesh=plsc.VectorSubcoreMesh(core_axis_name="c", subcore_axis_name="s"),
    compiler_params=pltpu.CompilerParams(needs_layout_passes=False),
    scratch_types=[
        pltpu.VMEM((_CHUNK,), jnp.int32),
        pltpu.VMEM((_CHUNK,), jnp.int32),
        pltpu.VMEM((_CHUNK,), jnp.int32),
        pltpu.VMEM((_CHUNK,), jnp.int32),
        pltpu.VMEM((_CHUNK, _D), jnp.float32),
        pltpu.VMEM((_CHUNK, _D), jnp.float32),
        pltpu.VMEM((_CHUNK, _D), jnp.float32),
        pltpu.VMEM((_CHUNK, _D), jnp.float32),
        pltpu.VMEM((_CPAD,), jnp.float32),
        pltpu.VMEM_SHARED((_CPAD, _D), jnp.float32),
        pltpu.SemaphoreType.DMA,
        pltpu.SemaphoreType.DMA,
    ],
)(_sc_body)


def _tc_body(sums_ref, cnts_ref, out_ref):
    sums = sums_ref[...]
    seg = sums[:_CPAD] + sums[_CPAD:]
    cnt = jnp.sum(cnts_ref[...], axis=0)[:, None]
    total = jnp.sum(seg, axis=0, keepdims=True)
    n = jnp.float32(_N)
    csafe = jnp.where(cnt > 0, cnt, 1.0)
    diff = seg / csafe - (total - seg) / (n - csafe) + 1e-6
    dist = jnp.sqrt(jnp.sum(diff * diff, axis=1, keepdims=True))
    term = jnp.square(jnp.maximum(_MARGIN - dist, 0.0))
    loss = jnp.sum(jnp.where(cnt > 0, cnt * term, 0.0)) / n
    out_ref[...] = jnp.reshape(loss, (1, 1))


def kernel(output, label_id):
    sums, cnts = _sc_segsum(output, label_id)
    loss = pl.pallas_call(
        _tc_body,
        out_shape=jax.ShapeDtypeStruct((1, 1), jnp.float32),
    )(sums, cnts)
    return loss.reshape(())
